# baseline (speedup 1.0000x reference)
import functools

import jax
import jax.numpy as jnp
from jax import lax
from jax.experimental import pallas as pl
from jax.experimental.pallas import tpu as pltpu
from jax.experimental.pallas import tpu_sc as plsc

IN_SIZE = 1024
HID = 1024
MEM_SIZE = 65536
MEM_DIM = 512
OUT_SIZE = 1024
TOP_K = 3
BLK = 4096
NBLK = MEM_SIZE // BLK
NEG_INF = float("-inf")
IMAX = 2**31 - 1

LANE = 16
NW = 32
TILE32 = MEM_SIZE // NW
SC_K = 3
NCAND = 16


def _tc1a_body(xlast_ref, W1_ref, b1_ref, W2_ref, b2_ref, Wq_ref, bq_ref,
               keys_ref, imp_ref, Wout_t_ref, bout_ref,
               wsims_ref, part_ref, qn_ref, qn_s):
    step = pl.program_id(0)

    @pl.when(step == 0)
    def _init():
        x = xlast_ref[...]
        h1 = jnp.maximum(
            jnp.dot(x, W1_ref[...], preferred_element_type=jnp.float32)
            + b1_ref[...], 0.0)
        h2 = jnp.dot(h1, W2_ref[...], preferred_element_type=jnp.float32) \
            + b2_ref[...]
        part_ref[...] = jnp.dot(h2, Wout_t_ref[...],
                                preferred_element_type=jnp.float32) \
            + bout_ref[...]
        q = jnp.dot(h2, Wq_ref[...], preferred_element_type=jnp.float32) \
            + bq_ref[...]
        qnorm = jnp.sqrt(jnp.sum(q * q))
        qn = q / jnp.maximum(qnorm, 1e-12)
        qn_ref[...] = qn
        qn_s[...] = qn

    _sims_step(keys_ref, imp_ref, qn_s, wsims_ref)


def _sims_step(keys_ref, imp_ref, qn_s, wsims_ref):
    blk = keys_ref[...]
    qn = qn_s[...]
    dn = (((1,), (1,)), ((), ()))
    dots = lax.dot_general(qn, blk, dn,
                           preferred_element_type=jnp.float32)
    sq = blk * blk
    ones = jnp.ones((1, MEM_DIM), dtype=jnp.float32)
    rn = lax.dot_general(ones, sq, dn,
                         preferred_element_type=jnp.float32)
    w = dots / jnp.maximum(jnp.sqrt(rn), 1e-12) * imp_ref[0]
    wsims_ref[...] = w.reshape(1, 1, BLK)


def _tc1b_body(qn_ref, keys_ref, imp_ref, wsims_ref, qn_s):
    @pl.when(pl.program_id(0) == 0)
    def _():
        qn_s[...] = qn_ref[...]

    _sims_step(keys_ref, imp_ref, qn_s, wsims_ref)


NHALF = NBLK // 2


def _tc1a(x_last, W1, b1, W2, b2, Wq, bq, keys_half, imp3_half, Wout_top, bout):
    full = lambda i: (0, 0)
    grid_spec = pltpu.PrefetchScalarGridSpec(
        num_scalar_prefetch=0,
        grid=(NHALF,),
        in_specs=[
            pl.BlockSpec((1, IN_SIZE), full),
            pl.BlockSpec((IN_SIZE, HID), full),
            pl.BlockSpec((1, HID), full),
            pl.BlockSpec((HID, HID), full),
            pl.BlockSpec((1, HID), full),
            pl.BlockSpec((HID, MEM_DIM), full),
            pl.BlockSpec((1, MEM_DIM), full),
            pl.BlockSpec((BLK, MEM_DIM), lambda i: (i, 0)),
            pl.BlockSpec((1, 1, BLK), lambda i: (i, 0, 0)),
            pl.BlockSpec((HID, OUT_SIZE), full),
            pl.BlockSpec((1, OUT_SIZE), full),
        ],
        out_specs=(
            pl.BlockSpec((1, 1, BLK), lambda i: (i, 0, 0)),
            pl.BlockSpec((1, OUT_SIZE), full),
            pl.BlockSpec((1, MEM_DIM), full),
        ),
        scratch_shapes=[pltpu.VMEM((1, MEM_DIM), jnp.float32)],
    )
    return pl.pallas_call(
        _tc1a_body,
        grid_spec=grid_spec,
        out_shape=(
            jax.ShapeDtypeStruct((NHALF, 1, BLK), jnp.float32),
            jax.ShapeDtypeStruct((1, OUT_SIZE), jnp.float32),
            jax.ShapeDtypeStruct((1, MEM_DIM), jnp.float32),
        ),
        compiler_params=pltpu.CompilerParams(
            dimension_semantics=("arbitrary",),
        ),
    )(x_last, W1, b1, W2, b2, Wq, bq, keys_half, imp3_half, Wout_top, bout)


def _tc1b(qn, keys_half, imp3_half):
    full = lambda i: (0, 0)
    grid_spec = pltpu.PrefetchScalarGridSpec(
        num_scalar_prefetch=0,
        grid=(NHALF,),
        in_specs=[
            pl.BlockSpec((1, MEM_DIM), full),
            pl.BlockSpec((BLK, MEM_DIM), lambda i: (i + NHALF, 0)),
            pl.BlockSpec((1, 1, BLK), lambda i: (i + NHALF, 0, 0)),
        ],
        out_specs=pl.BlockSpec((1, 1, BLK), lambda i: (i, 0, 0)),
        scratch_shapes=[pltpu.VMEM((1, MEM_DIM), jnp.float32)],
    )
    return pl.pallas_call(
        _tc1b_body,
        grid_spec=grid_spec,
        out_shape=jax.ShapeDtypeStruct((NHALF, 1, BLK), jnp.float32),
        compiler_params=pltpu.CompilerParams(
            dimension_semantics=("arbitrary",),
        ),
    )(qn, keys_half, imp3_half)


def _iota16():
    return lax.broadcasted_iota(jnp.int32, (LANE,), 0)


def _insert_top(v, i, tv, ti):
    k = len(tv)
    cs = [v > t for t in tv]
    nv, ni = list(tv), list(ti)
    for j in range(k - 1, 0, -1):
        nv[j] = jnp.where(cs[j - 1], tv[j - 1], jnp.where(cs[j], v, tv[j]))
        ni[j] = jnp.where(cs[j - 1], ti[j - 1], jnp.where(cs[j], i, ti[j]))
    nv[0] = jnp.where(cs[0], v, tv[0])
    ni[0] = jnp.where(cs[0], i, ti[0])
    return nv, ni


def _take16(v, idx):
    dn = lax.GatherDimensionNumbers(
        offset_dims=(), collapsed_slice_dims=(0,), start_index_map=(0,))
    return lax.gather(v, idx[:, None], dn, slice_sizes=(1,),
                      mode=lax.GatherScatterMode.PROMISE_IN_BOUNDS)


def _butterfly(v, op):
    it = _iota16()
    for k in (1, 2, 4, 8):
        v = op(v, _take16(v, jnp.bitwise_xor(it, k)))
    return v


def _bcast_max(v):
    return _butterfly(v, jnp.maximum)


def _bcast_min(v):
    return _butterfly(v, jnp.minimum)


def _extract_max(tv, ti):
    g = _bcast_max(tv[0])
    eq = tv[0] == g
    sel = _bcast_min(jnp.where(eq, ti[0], IMAX))
    rem = eq & (ti[0] == sel)
    nv, ni = list(tv), list(ti)
    for j in range(len(tv) - 1):
        nv[j] = jnp.where(rem, tv[j + 1], tv[j])
        ni[j] = jnp.where(rem, ti[j + 1], ti[j])
    nv[-1] = jnp.where(rem, NEG_INF, tv[-1])
    return g, sel, nv, ni


def _splats_to_vec(splats, fill, dtype):
    vec = jnp.full((LANE,), fill, dtype=dtype)
    it = _iota16()
    for j, s in enumerate(splats):
        vec = jnp.where(it == j, s, vec)
    return vec


HALF_N = MEM_SIZE // 2
TILEH = HALF_N // NW


def _sc_scan_body(base_off, wsims_hbm, vals_hbm, idxs_hbm,
                  sims_v, triple_v, triple_i):
    cid = lax.axis_index("c")
    sid = lax.axis_index("s")
    wid = sid * 2 + cid
    base = wid * TILEH
    pltpu.sync_copy(wsims_hbm.at[pl.ds(base, TILEH)], sims_v)
    it = _iota16()

    def scan_step(k, carry):
        tv = carry[0:SC_K]
        ti = carry[SC_K:2 * SC_K]
        v = sims_v[pl.ds(k * LANE, LANE)]
        idx = base_off + base + k * LANE + it
        tv, ti = _insert_top(v, idx, tv, ti)
        return tuple(tv) + tuple(ti)

    ninf = jnp.full((LANE,), NEG_INF, dtype=jnp.float32)
    zero = jnp.zeros((LANE,), dtype=jnp.int32)
    carry = lax.fori_loop(0, TILEH // LANE, scan_step,
                          (ninf,) * SC_K + (zero,) * SC_K)
    tv = list(carry[0:SC_K])
    ti = list(carry[SC_K:2 * SC_K])
    vals, idxs = [], []
    for _ in range(SC_K):
        g, sel, tv, ti = _extract_max(tv, ti)
        vals.append(g)
        idxs.append(sel)
    triple_v[...] = _splats_to_vec(vals, NEG_INF, jnp.float32)
    triple_i[...] = _splats_to_vec(idxs, 0, jnp.int32)
    pltpu.sync_copy(triple_v, vals_hbm.at[wid])
    pltpu.sync_copy(triple_i, idxs_hbm.at[wid])


def _make_sc_scan(base_off):
    @functools.partial(
        pl.kernel,
        mesh=plsc.VectorSubcoreMesh(core_axis_name="c", subcore_axis_name="s"),
        out_type=(
            jax.ShapeDtypeStruct((NW, LANE), jnp.float32),
            jax.ShapeDtypeStruct((NW, LANE), jnp.int32),
        ),
        scratch_types=[
            pltpu.VMEM((TILEH,), jnp.float32),
            pltpu.VMEM((LANE,), jnp.float32),
            pltpu.VMEM((LANE,), jnp.int32),
        ],
    )
    def scan(wsims_hbm, vals_hbm, idxs_hbm, *scratch):
        _sc_scan_body(base_off, wsims_hbm, vals_hbm, idxs_hbm, *scratch)

    return scan


_sc_scan_lo = _make_sc_scan(0)
_sc_scan_hi = _make_sc_scan(HALF_N)


def _tc2_body(vals_ref, idxs_ref, part_ref, Wout_b_ref, mem_vals_ref,
              out_ref, row_s, sem):
    V = vals_ref[...]
    I = idxs_ref[...]
    tv, ti, cps = [], [], []
    for j in range(TOP_K):
        m = jnp.max(V)
        sel = jnp.min(jnp.where(V == m, I, IMAX))
        V = jnp.where(I == sel, NEG_INF, V)
        tv.append(m)
        ti.append(sel)
        cp = pltpu.make_async_copy(
            mem_vals_ref.at[pl.ds(sel, 1)], row_s.at[pl.ds(j, 1)], sem)
        cp.start()
        cps.append(cp)
    for cp in cps:
        cp.wait()

    m0 = tv[0]
    e = [jnp.exp(jnp.full((1, MEM_DIM), tv[j] - m0, dtype=jnp.float32))
         for j in range(TOP_K)]
    den = e[0] + e[1] + e[2]
    retrieved = (e[0] * row_s[0:1, :] + e[1] * row_s[1:2, :]
                 + e[2] * row_s[2:3, :]) / den
    out_ref[...] = part_ref[...] + jnp.dot(
        retrieved, Wout_b_ref[...], preferred_element_type=jnp.float32)


def _tc2(vals, idxs, part, Wout_bot, mem_vals):
    return pl.pallas_call(
        _tc2_body,
        in_specs=[
            pl.BlockSpec((2 * NW, LANE), lambda: (0, 0)),
            pl.BlockSpec((2 * NW, LANE), lambda: (0, 0)),
            pl.BlockSpec((1, OUT_SIZE), lambda: (0, 0)),
            pl.BlockSpec((MEM_DIM, OUT_SIZE), lambda: (0, 0)),
            pl.BlockSpec(memory_space=pl.ANY),
        ],
        out_specs=pl.BlockSpec((1, OUT_SIZE), lambda: (0, 0)),
        out_shape=jax.ShapeDtypeStruct((1, OUT_SIZE), jnp.float32),
        scratch_shapes=[
            pltpu.VMEM((8, MEM_DIM), jnp.float32),
            pltpu.SemaphoreType.DMA,
        ],
    )(vals, idxs, part, Wout_bot, mem_vals)


def kernel(x, W1, b1, W2, b2, Wq, bq, mem_keys, mem_vals, importance, Wout, bout):
    x_last = x[:, -1, :]
    imp3 = importance.reshape(NBLK, 1, BLK)
    wsims_lo, part, qn = _tc1a(x_last, W1, b1.reshape(1, HID), W2,
                               b2.reshape(1, HID), Wq, bq.reshape(1, MEM_DIM),
                               mem_keys, imp3, Wout[:HID],
                               bout.reshape(1, OUT_SIZE))
    wsims_hi = _tc1b(qn, mem_keys, imp3)
    va, ia = _sc_scan_lo(wsims_lo.reshape(HALF_N))
    vb, ib = _sc_scan_hi(wsims_hi.reshape(HALF_N))
    vals = jnp.concatenate([va, vb], axis=0)
    idxs = jnp.concatenate([ia, ib], axis=0)
    return _tc2(vals, idxs, part, Wout[HID:], mem_vals)


# speedup vs baseline: 1.1585x; 1.1585x over previous
import functools

import jax
import jax.numpy as jnp
from jax import lax
from jax.experimental import pallas as pl
from jax.experimental.pallas import tpu as pltpu
from jax.experimental.pallas import tpu_sc as plsc

IN_SIZE = 1024
HID = 1024
MEM_SIZE = 65536
MEM_DIM = 512
OUT_SIZE = 1024
TOP_K = 3
BLK = 4096
NBLK = MEM_SIZE // BLK
NEG_INF = float("-inf")
IMAX = 2**31 - 1

LANE = 16
NW = 32
TILE32 = MEM_SIZE // NW
SC_K = 3
NCAND = 16


def _tc1_body(xlast_ref, W1_ref, b1_ref, W2_ref, b2_ref, Wq_ref, bq_ref,
              keys_ref, imp_ref, Wout_t_ref, bout_ref,
              wsims_ref, part_ref, qn_s):
    step = pl.program_id(0)

    @pl.when(step == 0)
    def _init():
        x = xlast_ref[...]
        h1 = jnp.maximum(
            jnp.dot(x, W1_ref[...], preferred_element_type=jnp.float32)
            + b1_ref[...], 0.0)
        h2 = jnp.dot(h1, W2_ref[...], preferred_element_type=jnp.float32) \
            + b2_ref[...]
        part_ref[...] = jnp.dot(h2, Wout_t_ref[...],
                                preferred_element_type=jnp.float32) \
            + bout_ref[...]
        q = jnp.dot(h2, Wq_ref[...], preferred_element_type=jnp.float32) \
            + bq_ref[...]
        qnorm = jnp.sqrt(jnp.sum(q * q))
        qn_s[...] = q / jnp.maximum(qnorm, 1e-12)

    blk = keys_ref[...]
    qn = qn_s[...]
    dn = (((1,), (1,)), ((), ()))
    dots = lax.dot_general(qn, blk, dn,
                           preferred_element_type=jnp.float32)
    sq = blk * blk
    ones = jnp.ones((1, MEM_DIM), dtype=jnp.float32)
    rn = lax.dot_general(ones, sq, dn,
                         preferred_element_type=jnp.float32)
    w = dots / jnp.maximum(jnp.sqrt(rn), 1e-12) * imp_ref[0]
    wsims_ref[...] = w.reshape(1, 1, BLK)


def _tc1(x_last, W1, b1, W2, b2, Wq, bq, mem_keys, imp3, Wout_top, bout):
    full = lambda i: (0, 0)
    grid_spec = pltpu.PrefetchScalarGridSpec(
        num_scalar_prefetch=0,
        grid=(NBLK,),
        in_specs=[
            pl.BlockSpec((1, IN_SIZE), full),
            pl.BlockSpec((IN_SIZE, HID), full),
            pl.BlockSpec((1, HID), full),
            pl.BlockSpec((HID, HID), full),
            pl.BlockSpec((1, HID), full),
            pl.BlockSpec((HID, MEM_DIM), full),
            pl.BlockSpec((1, MEM_DIM), full),
            pl.BlockSpec((BLK, MEM_DIM), lambda i: (i, 0)),
            pl.BlockSpec((1, 1, BLK), lambda i: (i, 0, 0)),
            pl.BlockSpec((HID, OUT_SIZE), full),
            pl.BlockSpec((1, OUT_SIZE), full),
        ],
        out_specs=(
            pl.BlockSpec((1, 1, BLK), lambda i: (i, 0, 0)),
            pl.BlockSpec((1, OUT_SIZE), full),
        ),
        scratch_shapes=[pltpu.VMEM((1, MEM_DIM), jnp.float32)],
    )
    return pl.pallas_call(
        _tc1_body,
        grid_spec=grid_spec,
        out_shape=(
            jax.ShapeDtypeStruct((NBLK, 1, BLK), jnp.float32),
            jax.ShapeDtypeStruct((1, OUT_SIZE), jnp.float32),
        ),
        compiler_params=pltpu.CompilerParams(
            dimension_semantics=("arbitrary",),
        ),
    )(x_last, W1, b1, W2, b2, Wq, bq, mem_keys, imp3, Wout_top, bout)


def _iota16():
    return lax.broadcasted_iota(jnp.int32, (LANE,), 0)


def _insert_top(v, i, tv, ti):
    k = len(tv)
    cs = [v > t for t in tv]
    nv, ni = list(tv), list(ti)
    for j in range(k - 1, 0, -1):
        nv[j] = jnp.where(cs[j - 1], tv[j - 1], jnp.where(cs[j], v, tv[j]))
        ni[j] = jnp.where(cs[j - 1], ti[j - 1], jnp.where(cs[j], i, ti[j]))
    nv[0] = jnp.where(cs[0], v, tv[0])
    ni[0] = jnp.where(cs[0], i, ti[0])
    return nv, ni


def _take16(v, idx):
    dn = lax.GatherDimensionNumbers(
        offset_dims=(), collapsed_slice_dims=(0,), start_index_map=(0,))
    return lax.gather(v, idx[:, None], dn, slice_sizes=(1,),
                      mode=lax.GatherScatterMode.PROMISE_IN_BOUNDS)


def _butterfly(v, op):
    it = _iota16()
    for k in (1, 2, 4, 8):
        v = op(v, _take16(v, jnp.bitwise_xor(it, k)))
    return v


def _bcast_max(v):
    return _butterfly(v, jnp.maximum)


def _bcast_min(v):
    return _butterfly(v, jnp.minimum)


def _extract_max(tv, ti):
    g = _bcast_max(tv[0])
    eq = tv[0] == g
    sel = _bcast_min(jnp.where(eq, ti[0], IMAX))
    rem = eq & (ti[0] == sel)
    nv, ni = list(tv), list(ti)
    for j in range(len(tv) - 1):
        nv[j] = jnp.where(rem, tv[j + 1], tv[j])
        ni[j] = jnp.where(rem, ti[j + 1], ti[j])
    nv[-1] = jnp.where(rem, NEG_INF, tv[-1])
    return g, sel, nv, ni


def _splats_to_vec(splats, fill, dtype):
    vec = jnp.full((LANE,), fill, dtype=dtype)
    it = _iota16()
    for j, s in enumerate(splats):
        vec = jnp.where(it == j, s, vec)
    return vec


def _sc_scan_body(wsims_hbm, vals_hbm, idxs_hbm, sims_v, triple_v, triple_i):
    cid = lax.axis_index("c")
    sid = lax.axis_index("s")
    wid = sid * 2 + cid
    base = wid * TILE32
    pltpu.sync_copy(wsims_hbm.at[pl.ds(base, TILE32)], sims_v)
    it = _iota16()

    def scan_step(k, carry):
        tv = carry[0:SC_K]
        ti = carry[SC_K:2 * SC_K]
        v = sims_v[pl.ds(k * LANE, LANE)]
        idx = base + k * LANE + it
        tv, ti = _insert_top(v, idx, tv, ti)
        return tuple(tv) + tuple(ti)

    ninf = jnp.full((LANE,), NEG_INF, dtype=jnp.float32)
    zero = jnp.zeros((LANE,), dtype=jnp.int32)
    carry = lax.fori_loop(0, TILE32 // LANE, scan_step,
                          (ninf,) * SC_K + (zero,) * SC_K)
    tv = list(carry[0:SC_K])
    ti = list(carry[SC_K:2 * SC_K])
    vals, idxs = [], []
    for _ in range(SC_K):
        g, sel, tv, ti = _extract_max(tv, ti)
        vals.append(g)
        idxs.append(sel)
    triple_v[...] = _splats_to_vec(vals, NEG_INF, jnp.float32)
    triple_i[...] = _splats_to_vec(idxs, 0, jnp.int32)
    pltpu.sync_copy(triple_v, vals_hbm.at[wid])
    pltpu.sync_copy(triple_i, idxs_hbm.at[wid])


@functools.partial(
    pl.kernel,
    mesh=plsc.VectorSubcoreMesh(core_axis_name="c", subcore_axis_name="s"),
    out_type=(
        jax.ShapeDtypeStruct((NW, LANE), jnp.float32),
        jax.ShapeDtypeStruct((NW, LANE), jnp.int32),
    ),
    scratch_types=[
        pltpu.VMEM((TILE32,), jnp.float32),
        pltpu.VMEM((LANE,), jnp.float32),
        pltpu.VMEM((LANE,), jnp.int32),
    ],
)
def _sc_scan(wsims_hbm, vals_hbm, idxs_hbm, *scratch):
    _sc_scan_body(wsims_hbm, vals_hbm, idxs_hbm, *scratch)


def _tc2_body(vals_ref, idxs_ref, part_ref, Wout_b_ref, mem_vals_ref,
              out_ref, row_s, sem):
    V = vals_ref[...]
    I = idxs_ref[...]
    tv, ti, cps = [], [], []
    for j in range(TOP_K):
        m = jnp.max(V)
        sel = jnp.min(jnp.where(V == m, I, IMAX))
        V = jnp.where(I == sel, NEG_INF, V)
        tv.append(m)
        ti.append(sel)
        cp = pltpu.make_async_copy(
            mem_vals_ref.at[pl.ds(sel, 1)], row_s.at[pl.ds(j, 1)], sem)
        cp.start()
        cps.append(cp)
    for cp in cps:
        cp.wait()

    m0 = tv[0]
    e = [jnp.exp(jnp.full((1, MEM_DIM), tv[j] - m0, dtype=jnp.float32))
         for j in range(TOP_K)]
    den = e[0] + e[1] + e[2]
    retrieved = (e[0] * row_s[0:1, :] + e[1] * row_s[1:2, :]
                 + e[2] * row_s[2:3, :]) / den
    out_ref[...] = part_ref[...] + jnp.dot(
        retrieved, Wout_b_ref[...], preferred_element_type=jnp.float32)


def _tc2(vals, idxs, part, Wout_bot, mem_vals):
    return pl.pallas_call(
        _tc2_body,
        in_specs=[
            pl.BlockSpec((NW, LANE), lambda: (0, 0)),
            pl.BlockSpec((NW, LANE), lambda: (0, 0)),
            pl.BlockSpec((1, OUT_SIZE), lambda: (0, 0)),
            pl.BlockSpec((MEM_DIM, OUT_SIZE), lambda: (0, 0)),
            pl.BlockSpec(memory_space=pl.ANY),
        ],
        out_specs=pl.BlockSpec((1, OUT_SIZE), lambda: (0, 0)),
        out_shape=jax.ShapeDtypeStruct((1, OUT_SIZE), jnp.float32),
        scratch_shapes=[
            pltpu.VMEM((8, MEM_DIM), jnp.float32),
            pltpu.SemaphoreType.DMA,
        ],
    )(vals, idxs, part, Wout_bot, mem_vals)


def kernel(x, W1, b1, W2, b2, Wq, bq, mem_keys, mem_vals, importance, Wout, bout):
    x_last = x[:, -1, :]
    imp3 = importance.reshape(NBLK, 1, BLK)
    wsims, part = _tc1(x_last, W1, b1.reshape(1, HID), W2,
                       b2.reshape(1, HID), Wq, bq.reshape(1, MEM_DIM),
                       mem_keys, imp3, Wout[:HID],
                       bout.reshape(1, OUT_SIZE))
    vals, idxs = _sc_scan(wsims.reshape(MEM_SIZE))
    return _tc2(vals, idxs, part, Wout[HID:], mem_vals)
